# baseline (speedup 1.0000x reference)
import functools

import jax
import jax.numpy as jnp
from jax import lax
from jax.experimental import pallas as pl
from jax.experimental.pallas import tpu as pltpu
from jax.experimental.pallas import tpu_sc as plsc

BATCH = 4096
SEQ = 200
DIM = 64
HID = 128
OUT = 10
VOCAB1 = 1000001

NC = 2
NS = 16
NW = NC * NS
BPW = BATCH // NW
LANES = 16
CH0 = 128
CH1 = SEQ - CH0

TBLK = 512
NTB = (VOCAB1 + TBLK - 1) // TBLK
PADV = NTB * TBLK


def _transpose_body(xt_ref, o_ref):
    x = xt_ref[:]
    t = jnp.transpose(x, (1, 0))
    o_ref[:] = jnp.concatenate(
        [t, jnp.zeros((TBLK, 128 - DIM), jnp.float32)], axis=1)


@jax.jit
def _widen(tableT):
    return pl.pallas_call(
        _transpose_body,
        grid=(NTB,),
        in_specs=[pl.BlockSpec((DIM, TBLK), lambda i: (0, i))],
        out_specs=pl.BlockSpec((TBLK, 128), lambda i: (i, 0)),
        out_shape=jax.ShapeDtypeStruct((PADV, 128), jnp.float32),
    )(tableT)


NBUF = 3


def _fire_row(table_hbm, idx_v, buf, sem, r):
    pltpu.async_copy(table_hbm.at[idx_v.at[r, pl.ds(0, CH0)]],
                     buf.at[pl.ds(0, CH0), :], sem)
    pltpu.async_copy(table_hbm.at[idx_v.at[r, pl.ds(CH0, CH1)]],
                     buf.at[pl.ds(CH0, CH1), :], sem)


def _drain_row(table_hbm, buf, sem):
    pltpu.make_async_copy(table_hbm.at[pl.ds(0, SEQ)], buf, sem).wait()


def _pool_body(idx_hbm, table_hbm, out_hbm, idx_v,
               buf0, buf1, buf2, acc_v, sem0, sem1, sem2):
    c = lax.axis_index("c")
    s = lax.axis_index("s")
    wid = s * NC + c
    base = wid * BPW

    pltpu.sync_copy(idx_hbm.at[pl.ds(base, BPW), :], idx_v)

    bufs = (buf0, buf1, buf2)
    sems = (sem0, sem1, sem2)
    for b in range(NBUF - 1):
        _fire_row(table_hbm, idx_v, bufs[b], sems[b], b)

    zero = jnp.zeros((LANES,), jnp.float32)

    def accum_row(buf, r):
        def lbody(l, carry):
            a0, a1, a2, a3 = carry
            a0 = a0 + buf[l, pl.ds(0, LANES)]
            a1 = a1 + buf[l, pl.ds(LANES, LANES)]
            a2 = a2 + buf[l, pl.ds(2 * LANES, LANES)]
            a3 = a3 + buf[l, pl.ds(3 * LANES, LANES)]
            return (a0, a1, a2, a3)

        a0, a1, a2, a3 = lax.fori_loop(0, SEQ, lbody, (zero,) * 4,
                                       unroll=8)
        acc_v[r, pl.ds(0, LANES)] = a0
        acc_v[r, pl.ds(LANES, LANES)] = a1
        acc_v[r, pl.ds(2 * LANES, LANES)] = a2
        acc_v[r, pl.ds(3 * LANES, LANES)] = a3

    def do_row(r, b):
        @pl.when(r + NBUF - 1 < BPW)
        def _():
            _fire_row(table_hbm, idx_v, bufs[(b + NBUF - 1) % NBUF],
                      sems[(b + NBUF - 1) % NBUF], r + NBUF - 1)

        _drain_row(table_hbm, bufs[b], sems[b])
        accum_row(bufs[b], r)

    NG = BPW // NBUF

    def step(g, _):
        for b in range(NBUF):
            do_row(g * NBUF + b, b)
        return 0

    lax.fori_loop(0, NG, step, 0)
    for r in range(NG * NBUF, BPW):
        do_row(r, r % NBUF)

    pltpu.sync_copy(acc_v, out_hbm.at[pl.ds(base, BPW)])


@jax.jit
def _pool(idx, table):
    mesh = plsc.VectorSubcoreMesh(core_axis_name="c", subcore_axis_name="s")
    return pl.kernel(
        _pool_body,
        mesh=mesh,
        compiler_params=pltpu.CompilerParams(use_tc_tiling_on_sc=True),
        out_type=jax.ShapeDtypeStruct((BATCH, DIM), jnp.float32),
        scratch_types=[
            pltpu.VMEM((BPW, SEQ), jnp.int32),
            pltpu.VMEM((SEQ, 128), jnp.float32),
            pltpu.VMEM((SEQ, 128), jnp.float32),
            pltpu.VMEM((SEQ, 128), jnp.float32),
            pltpu.VMEM((BPW, DIM), jnp.float32),
            pltpu.SemaphoreType.DMA,
            pltpu.SemaphoreType.DMA,
            pltpu.SemaphoreType.DMA,
        ],
    )(idx, table)


def _mlp_body(x_ref, w1_ref, b1_ref, w2_ref, b2_ref, o_ref):
    x = x_ref[:] * (1.0 / SEQ)
    h = jnp.dot(x, w1_ref[:], preferred_element_type=jnp.float32) + b1_ref[:]
    z = jnp.dot(h, w2_ref[:], preferred_element_type=jnp.float32) + b2_ref[:]
    z = z - jnp.max(z, axis=-1, keepdims=True)
    e = jnp.exp(z)
    o_ref[:] = e / jnp.sum(e, axis=-1, keepdims=True)


@jax.jit
def _mlp(pooled, W1, b1, W2, b2):
    blk = 1024
    grid = BATCH // blk
    return pl.pallas_call(
        _mlp_body,
        grid=(grid,),
        in_specs=[
            pl.BlockSpec((blk, DIM), lambda i: (i, 0)),
            pl.BlockSpec((DIM, HID), lambda i: (0, 0)),
            pl.BlockSpec((1, HID), lambda i: (0, 0)),
            pl.BlockSpec((HID, OUT), lambda i: (0, 0)),
            pl.BlockSpec((1, OUT), lambda i: (0, 0)),
        ],
        out_specs=pl.BlockSpec((blk, OUT), lambda i: (i, 0)),
        out_shape=jax.ShapeDtypeStruct((BATCH, OUT), jnp.float32),
    )(pooled, W1, b1, W2, b2)


def kernel(inputs, emb_table, W1, b1, W2, b2):
    idx = inputs.astype(jnp.int32)
    table128 = _widen(emb_table.T)
    pooled = _pool(idx, table128)
    return _mlp(pooled, W1, b1.reshape(1, HID), W2, b2.reshape(1, OUT))


# speedup vs baseline: 2.6112x; 2.6112x over previous
import functools

import jax
import jax.numpy as jnp
from jax import lax
from jax.experimental import pallas as pl
from jax.experimental.pallas import tpu as pltpu
from jax.experimental.pallas import tpu_sc as plsc

BATCH = 4096
SEQ = 200
DIM = 64
HID = 128
OUT = 10
VOCAB1 = 1000001

NC = 2
NS = 16
NW = NC * NS
BPW = BATCH // NW
LANES = 16
CH0 = 128
CH1 = SEQ - CH0

TBLK = 4096
NTB = (VOCAB1 + TBLK - 1) // TBLK
PADV = NTB * TBLK


def _transpose_body(xt_ref, o_ref):
    x = xt_ref[:]
    t = jnp.transpose(x, (1, 0))
    o_ref[:] = jnp.concatenate(
        [t, jnp.zeros((TBLK, 128 - DIM), jnp.float32)], axis=1)


@jax.jit
def _widen(tableT):
    return pl.pallas_call(
        _transpose_body,
        grid=(NTB,),
        in_specs=[pl.BlockSpec((DIM, TBLK), lambda i: (0, i))],
        out_specs=pl.BlockSpec((TBLK, 128), lambda i: (i, 0)),
        out_shape=jax.ShapeDtypeStruct((PADV, 128), jnp.float32),
    )(tableT)


NBUF = 3


def _fire_row(table_hbm, idx_v, buf, sem, r):
    pltpu.async_copy(table_hbm.at[idx_v.at[r, pl.ds(0, CH0)]],
                     buf.at[pl.ds(0, CH0), :], sem)
    pltpu.async_copy(table_hbm.at[idx_v.at[r, pl.ds(CH0, CH1)]],
                     buf.at[pl.ds(CH0, CH1), :], sem)


def _drain_row(table_hbm, buf, sem):
    pltpu.make_async_copy(table_hbm.at[pl.ds(0, SEQ)], buf, sem).wait()


def _pool_body(idx_hbm, table_hbm, out_hbm, idx_v,
               buf0, buf1, buf2, acc_v, sem0, sem1, sem2):
    c = lax.axis_index("c")
    s = lax.axis_index("s")
    wid = s * NC + c
    base = wid * BPW

    pltpu.sync_copy(idx_hbm.at[pl.ds(base, BPW), :], idx_v)

    bufs = (buf0, buf1, buf2)
    sems = (sem0, sem1, sem2)
    for b in range(NBUF - 1):
        _fire_row(table_hbm, idx_v, bufs[b], sems[b], b)

    zero = jnp.zeros((LANES,), jnp.float32)

    def accum_row(buf, r):
        def lbody(l, carry):
            a0, a1, a2, a3 = carry
            a0 = a0 + buf[l, pl.ds(0, LANES)]
            a1 = a1 + buf[l, pl.ds(LANES, LANES)]
            a2 = a2 + buf[l, pl.ds(2 * LANES, LANES)]
            a3 = a3 + buf[l, pl.ds(3 * LANES, LANES)]
            return (a0, a1, a2, a3)

        a0, a1, a2, a3 = lax.fori_loop(0, SEQ, lbody, (zero,) * 4,
                                       unroll=8)
        acc_v[r, pl.ds(0, LANES)] = a0
        acc_v[r, pl.ds(LANES, LANES)] = a1
        acc_v[r, pl.ds(2 * LANES, LANES)] = a2
        acc_v[r, pl.ds(3 * LANES, LANES)] = a3

    def do_row(r, b):
        @pl.when(r + NBUF - 1 < BPW)
        def _():
            _fire_row(table_hbm, idx_v, bufs[(b + NBUF - 1) % NBUF],
                      sems[(b + NBUF - 1) % NBUF], r + NBUF - 1)

        _drain_row(table_hbm, bufs[b], sems[b])
        accum_row(bufs[b], r)

    NG = BPW // NBUF

    def step(g, _):
        for b in range(NBUF):
            do_row(g * NBUF + b, b)
        return 0

    lax.fori_loop(0, NG, step, 0)
    for r in range(NG * NBUF, BPW):
        do_row(r, r % NBUF)

    pltpu.sync_copy(acc_v, out_hbm.at[pl.ds(base, BPW)])


@jax.jit
def _pool(idx, table):
    mesh = plsc.VectorSubcoreMesh(core_axis_name="c", subcore_axis_name="s")
    return pl.kernel(
        _pool_body,
        mesh=mesh,
        compiler_params=pltpu.CompilerParams(use_tc_tiling_on_sc=True),
        out_type=jax.ShapeDtypeStruct((BATCH, DIM), jnp.float32),
        scratch_types=[
            pltpu.VMEM((BPW, SEQ), jnp.int32),
            pltpu.VMEM((SEQ, 128), jnp.float32),
            pltpu.VMEM((SEQ, 128), jnp.float32),
            pltpu.VMEM((SEQ, 128), jnp.float32),
            pltpu.VMEM((BPW, DIM), jnp.float32),
            pltpu.SemaphoreType.DMA,
            pltpu.SemaphoreType.DMA,
            pltpu.SemaphoreType.DMA,
        ],
    )(idx, table)


def _mlp_body(x_ref, w1_ref, b1_ref, w2_ref, b2_ref, o_ref):
    x = x_ref[:] * (1.0 / SEQ)
    h = jnp.dot(x, w1_ref[:], preferred_element_type=jnp.float32) + b1_ref[:]
    z = jnp.dot(h, w2_ref[:], preferred_element_type=jnp.float32) + b2_ref[:]
    z = z - jnp.max(z, axis=-1, keepdims=True)
    e = jnp.exp(z)
    o_ref[:] = e / jnp.sum(e, axis=-1, keepdims=True)


@jax.jit
def _mlp(pooled, W1, b1, W2, b2):
    blk = 1024
    grid = BATCH // blk
    return pl.pallas_call(
        _mlp_body,
        grid=(grid,),
        in_specs=[
            pl.BlockSpec((blk, DIM), lambda i: (i, 0)),
            pl.BlockSpec((DIM, HID), lambda i: (0, 0)),
            pl.BlockSpec((1, HID), lambda i: (0, 0)),
            pl.BlockSpec((HID, OUT), lambda i: (0, 0)),
            pl.BlockSpec((1, OUT), lambda i: (0, 0)),
        ],
        out_specs=pl.BlockSpec((blk, OUT), lambda i: (i, 0)),
        out_shape=jax.ShapeDtypeStruct((BATCH, OUT), jnp.float32),
    )(pooled, W1, b1, W2, b2)


def kernel(inputs, emb_table, W1, b1, W2, b2):
    idx = inputs.astype(jnp.int32)
    table128 = _widen(emb_table.T)
    pooled = _pool(idx, table128)
    return _mlp(pooled, W1, b1.reshape(1, HID), W2, b2.reshape(1, OUT))


# speedup vs baseline: 3.0317x; 1.1611x over previous
import functools

import jax
import jax.numpy as jnp
from jax import lax
from jax.experimental import pallas as pl
from jax.experimental.pallas import tpu as pltpu
from jax.experimental.pallas import tpu_sc as plsc

BATCH = 4096
SEQ = 200
DIM = 64
HID = 128
OUT = 10
VOCAB1 = 1000001

NC = 2
NS = 16
NW = NC * NS
BPW = BATCH // NW
LANES = 16
CH0 = 128
CH1 = SEQ - CH0

TBLK = 8192
NTB = (VOCAB1 + TBLK - 1) // TBLK
PADV = NTB * TBLK


def _transpose_body(xt_ref, o_ref):
    x = xt_ref[:]
    t = jnp.transpose(x, (1, 0))
    o_ref[:] = jnp.concatenate(
        [t, jnp.zeros((TBLK, 128 - DIM), jnp.float32)], axis=1)


@jax.jit
def _widen(tableT):
    return pl.pallas_call(
        _transpose_body,
        grid=(NTB,),
        in_specs=[pl.BlockSpec((DIM, TBLK), lambda i: (0, i))],
        out_specs=pl.BlockSpec((TBLK, 128), lambda i: (i, 0)),
        out_shape=jax.ShapeDtypeStruct((PADV, 128), jnp.float32),
    )(tableT)


NBUF = 3


def _fire_row(table_hbm, idx_v, buf, sem, r):
    pltpu.async_copy(table_hbm.at[idx_v.at[r, pl.ds(0, CH0)]],
                     buf.at[pl.ds(0, CH0), :], sem)
    pltpu.async_copy(table_hbm.at[idx_v.at[r, pl.ds(CH0, CH1)]],
                     buf.at[pl.ds(CH0, CH1), :], sem)


def _drain_row(table_hbm, buf, sem):
    pltpu.make_async_copy(table_hbm.at[pl.ds(0, SEQ)], buf, sem).wait()


def _pool_body(idx_hbm, table_hbm, out_hbm, idx_v,
               buf0, buf1, buf2, acc_v, sem0, sem1, sem2):
    c = lax.axis_index("c")
    s = lax.axis_index("s")
    wid = s * NC + c
    base = wid * BPW

    pltpu.sync_copy(idx_hbm.at[pl.ds(base, BPW), :], idx_v)

    bufs = (buf0, buf1, buf2)
    sems = (sem0, sem1, sem2)
    for b in range(NBUF - 1):
        _fire_row(table_hbm, idx_v, bufs[b], sems[b], b)

    zero = jnp.zeros((LANES,), jnp.float32)

    def accum_row(buf, r):
        def lbody(l, carry):
            a0, a1, a2, a3 = carry
            a0 = a0 + buf[l, pl.ds(0, LANES)]
            a1 = a1 + buf[l, pl.ds(LANES, LANES)]
            a2 = a2 + buf[l, pl.ds(2 * LANES, LANES)]
            a3 = a3 + buf[l, pl.ds(3 * LANES, LANES)]
            return (a0, a1, a2, a3)

        a0, a1, a2, a3 = lax.fori_loop(0, SEQ, lbody, (zero,) * 4,
                                       unroll=8)
        acc_v[r, pl.ds(0, LANES)] = a0
        acc_v[r, pl.ds(LANES, LANES)] = a1
        acc_v[r, pl.ds(2 * LANES, LANES)] = a2
        acc_v[r, pl.ds(3 * LANES, LANES)] = a3

    def do_row(r, b):
        @pl.when(r + NBUF - 1 < BPW)
        def _():
            _fire_row(table_hbm, idx_v, bufs[(b + NBUF - 1) % NBUF],
                      sems[(b + NBUF - 1) % NBUF], r + NBUF - 1)

        _drain_row(table_hbm, bufs[b], sems[b])
        accum_row(bufs[b], r)

    NG = BPW // NBUF

    def step(g, _):
        for b in range(NBUF):
            do_row(g * NBUF + b, b)
        return 0

    lax.fori_loop(0, NG, step, 0)
    for r in range(NG * NBUF, BPW):
        do_row(r, r % NBUF)

    pltpu.sync_copy(acc_v, out_hbm.at[pl.ds(base, BPW)])


@jax.jit
def _pool(idx, table):
    mesh = plsc.VectorSubcoreMesh(core_axis_name="c", subcore_axis_name="s")
    return pl.kernel(
        _pool_body,
        mesh=mesh,
        compiler_params=pltpu.CompilerParams(use_tc_tiling_on_sc=True),
        out_type=jax.ShapeDtypeStruct((BATCH, DIM), jnp.float32),
        scratch_types=[
            pltpu.VMEM((BPW, SEQ), jnp.int32),
            pltpu.VMEM((SEQ, 128), jnp.float32),
            pltpu.VMEM((SEQ, 128), jnp.float32),
            pltpu.VMEM((SEQ, 128), jnp.float32),
            pltpu.VMEM((BPW, DIM), jnp.float32),
            pltpu.SemaphoreType.DMA,
            pltpu.SemaphoreType.DMA,
            pltpu.SemaphoreType.DMA,
        ],
    )(idx, table)


def _mlp_body(x_ref, w1_ref, b1_ref, w2_ref, b2_ref, o_ref):
    x = x_ref[:] * (1.0 / SEQ)
    h = jnp.dot(x, w1_ref[:], preferred_element_type=jnp.float32) + b1_ref[:]
    z = jnp.dot(h, w2_ref[:], preferred_element_type=jnp.float32) + b2_ref[:]
    z = z - jnp.max(z, axis=-1, keepdims=True)
    e = jnp.exp(z)
    o_ref[:] = e / jnp.sum(e, axis=-1, keepdims=True)


@jax.jit
def _mlp(pooled, W1, b1, W2, b2):
    blk = 1024
    grid = BATCH // blk
    return pl.pallas_call(
        _mlp_body,
        grid=(grid,),
        in_specs=[
            pl.BlockSpec((blk, DIM), lambda i: (i, 0)),
            pl.BlockSpec((DIM, HID), lambda i: (0, 0)),
            pl.BlockSpec((1, HID), lambda i: (0, 0)),
            pl.BlockSpec((HID, OUT), lambda i: (0, 0)),
            pl.BlockSpec((1, OUT), lambda i: (0, 0)),
        ],
        out_specs=pl.BlockSpec((blk, OUT), lambda i: (i, 0)),
        out_shape=jax.ShapeDtypeStruct((BATCH, OUT), jnp.float32),
    )(pooled, W1, b1, W2, b2)


def kernel(inputs, emb_table, W1, b1, W2, b2):
    idx = inputs.astype(jnp.int32)
    table128 = _widen(emb_table.T)
    pooled = _pool(idx, table128)
    return _mlp(pooled, W1, b1.reshape(1, HID), W2, b2.reshape(1, OUT))


# speedup vs baseline: 3.1486x; 1.0386x over previous
import functools

import jax
import jax.numpy as jnp
from jax import lax
from jax.experimental import pallas as pl
from jax.experimental.pallas import tpu as pltpu
from jax.experimental.pallas import tpu_sc as plsc

BATCH = 4096
SEQ = 200
DIM = 64
HID = 128
OUT = 10
VOCAB1 = 1000001

NC = 2
NS = 16
NW = NC * NS
BPW = BATCH // NW
LANES = 16
CH0 = 128
CH1 = SEQ - CH0

TBLK = 16384
NTB = (VOCAB1 + TBLK - 1) // TBLK
PADV = NTB * TBLK


def _transpose_body(xt_ref, o_ref):
    x = xt_ref[:]
    t = jnp.transpose(x, (1, 0))
    o_ref[:] = jnp.concatenate(
        [t, jnp.zeros((TBLK, 128 - DIM), jnp.float32)], axis=1)


@jax.jit
def _widen(tableT):
    return pl.pallas_call(
        _transpose_body,
        grid=(NTB,),
        in_specs=[pl.BlockSpec((DIM, TBLK), lambda i: (0, i))],
        out_specs=pl.BlockSpec((TBLK, 128), lambda i: (i, 0)),
        out_shape=jax.ShapeDtypeStruct((PADV, 128), jnp.float32),
    )(tableT)


NBUF = 3


def _fire_row(table_hbm, idx_v, buf, sem, r):
    pltpu.async_copy(table_hbm.at[idx_v.at[r, pl.ds(0, CH0)]],
                     buf.at[pl.ds(0, CH0), :], sem)
    pltpu.async_copy(table_hbm.at[idx_v.at[r, pl.ds(CH0, CH1)]],
                     buf.at[pl.ds(CH0, CH1), :], sem)


def _drain_row(table_hbm, buf, sem):
    pltpu.make_async_copy(table_hbm.at[pl.ds(0, SEQ)], buf, sem).wait()


def _pool_body(idx_hbm, table_hbm, out_hbm, idx_v,
               buf0, buf1, buf2, acc_v, sem0, sem1, sem2):
    c = lax.axis_index("c")
    s = lax.axis_index("s")
    wid = s * NC + c
    base = wid * BPW

    pltpu.sync_copy(idx_hbm.at[pl.ds(base, BPW), :], idx_v)

    bufs = (buf0, buf1, buf2)
    sems = (sem0, sem1, sem2)
    for b in range(NBUF - 1):
        _fire_row(table_hbm, idx_v, bufs[b], sems[b], b)

    zero = jnp.zeros((LANES,), jnp.float32)

    def accum_row(buf, r):
        def lbody(l, carry):
            a0, a1, a2, a3 = carry
            a0 = a0 + buf[l, pl.ds(0, LANES)]
            a1 = a1 + buf[l, pl.ds(LANES, LANES)]
            a2 = a2 + buf[l, pl.ds(2 * LANES, LANES)]
            a3 = a3 + buf[l, pl.ds(3 * LANES, LANES)]
            return (a0, a1, a2, a3)

        a0, a1, a2, a3 = lax.fori_loop(0, SEQ, lbody, (zero,) * 4,
                                       unroll=8)
        acc_v[r, pl.ds(0, LANES)] = a0
        acc_v[r, pl.ds(LANES, LANES)] = a1
        acc_v[r, pl.ds(2 * LANES, LANES)] = a2
        acc_v[r, pl.ds(3 * LANES, LANES)] = a3

    def do_row(r, b):
        @pl.when(r + NBUF - 1 < BPW)
        def _():
            _fire_row(table_hbm, idx_v, bufs[(b + NBUF - 1) % NBUF],
                      sems[(b + NBUF - 1) % NBUF], r + NBUF - 1)

        _drain_row(table_hbm, bufs[b], sems[b])
        accum_row(bufs[b], r)

    NG = BPW // NBUF

    def step(g, _):
        for b in range(NBUF):
            do_row(g * NBUF + b, b)
        return 0

    lax.fori_loop(0, NG, step, 0)
    for r in range(NG * NBUF, BPW):
        do_row(r, r % NBUF)

    pltpu.sync_copy(acc_v, out_hbm.at[pl.ds(base, BPW)])


@jax.jit
def _pool(idx, table):
    mesh = plsc.VectorSubcoreMesh(core_axis_name="c", subcore_axis_name="s")
    return pl.kernel(
        _pool_body,
        mesh=mesh,
        compiler_params=pltpu.CompilerParams(use_tc_tiling_on_sc=True),
        out_type=jax.ShapeDtypeStruct((BATCH, DIM), jnp.float32),
        scratch_types=[
            pltpu.VMEM((BPW, SEQ), jnp.int32),
            pltpu.VMEM((SEQ, 128), jnp.float32),
            pltpu.VMEM((SEQ, 128), jnp.float32),
            pltpu.VMEM((SEQ, 128), jnp.float32),
            pltpu.VMEM((BPW, DIM), jnp.float32),
            pltpu.SemaphoreType.DMA,
            pltpu.SemaphoreType.DMA,
            pltpu.SemaphoreType.DMA,
        ],
    )(idx, table)


def _mlp_body(x_ref, w1_ref, b1_ref, w2_ref, b2_ref, o_ref):
    x = x_ref[:] * (1.0 / SEQ)
    h = jnp.dot(x, w1_ref[:], preferred_element_type=jnp.float32) + b1_ref[:]
    z = jnp.dot(h, w2_ref[:], preferred_element_type=jnp.float32) + b2_ref[:]
    z = z - jnp.max(z, axis=-1, keepdims=True)
    e = jnp.exp(z)
    o_ref[:] = e / jnp.sum(e, axis=-1, keepdims=True)


@jax.jit
def _mlp(pooled, W1, b1, W2, b2):
    blk = 1024
    grid = BATCH // blk
    return pl.pallas_call(
        _mlp_body,
        grid=(grid,),
        in_specs=[
            pl.BlockSpec((blk, DIM), lambda i: (i, 0)),
            pl.BlockSpec((DIM, HID), lambda i: (0, 0)),
            pl.BlockSpec((1, HID), lambda i: (0, 0)),
            pl.BlockSpec((HID, OUT), lambda i: (0, 0)),
            pl.BlockSpec((1, OUT), lambda i: (0, 0)),
        ],
        out_specs=pl.BlockSpec((blk, OUT), lambda i: (i, 0)),
        out_shape=jax.ShapeDtypeStruct((BATCH, OUT), jnp.float32),
    )(pooled, W1, b1, W2, b2)


def kernel(inputs, emb_table, W1, b1, W2, b2):
    idx = inputs.astype(jnp.int32)
    table128 = _widen(emb_table.T)
    pooled = _pool(idx, table128)
    return _mlp(pooled, W1, b1.reshape(1, HID), W2, b2.reshape(1, OUT))


# speedup vs baseline: 3.2112x; 1.0199x over previous
import functools

import jax
import jax.numpy as jnp
from jax import lax
from jax.experimental import pallas as pl
from jax.experimental.pallas import tpu as pltpu
from jax.experimental.pallas import tpu_sc as plsc

BATCH = 4096
SEQ = 200
DIM = 64
HID = 128
OUT = 10
VOCAB1 = 1000001

NC = 2
NS = 16
NW = NC * NS
BPW = BATCH // NW
LANES = 16
CH0 = 128
CH1 = SEQ - CH0

TBLK = 32768
NTB = (VOCAB1 + TBLK - 1) // TBLK
PADV = NTB * TBLK


def _transpose_body(xt_ref, o_ref):
    x = xt_ref[:]
    t = jnp.transpose(x, (1, 0))
    o_ref[:] = jnp.concatenate(
        [t, jnp.zeros((TBLK, 128 - DIM), jnp.float32)], axis=1)


@jax.jit
def _widen(tableT):
    return pl.pallas_call(
        _transpose_body,
        grid=(NTB,),
        in_specs=[pl.BlockSpec((DIM, TBLK), lambda i: (0, i))],
        out_specs=pl.BlockSpec((TBLK, 128), lambda i: (i, 0)),
        out_shape=jax.ShapeDtypeStruct((PADV, 128), jnp.float32),
    )(tableT)


NBUF = 3


def _fire_row(table_hbm, idx_v, buf, sem, r):
    pltpu.async_copy(table_hbm.at[idx_v.at[r, pl.ds(0, CH0)]],
                     buf.at[pl.ds(0, CH0), :], sem)
    pltpu.async_copy(table_hbm.at[idx_v.at[r, pl.ds(CH0, CH1)]],
                     buf.at[pl.ds(CH0, CH1), :], sem)


def _drain_row(table_hbm, buf, sem):
    pltpu.make_async_copy(table_hbm.at[pl.ds(0, SEQ)], buf, sem).wait()


def _pool_body(idx_hbm, table_hbm, out_hbm, idx_v,
               buf0, buf1, buf2, acc_v, sem0, sem1, sem2):
    c = lax.axis_index("c")
    s = lax.axis_index("s")
    wid = s * NC + c
    base = wid * BPW

    pltpu.sync_copy(idx_hbm.at[pl.ds(base, BPW), :], idx_v)

    bufs = (buf0, buf1, buf2)
    sems = (sem0, sem1, sem2)
    for b in range(NBUF - 1):
        _fire_row(table_hbm, idx_v, bufs[b], sems[b], b)

    zero = jnp.zeros((LANES,), jnp.float32)

    def accum_row(buf, r):
        def lbody(l, carry):
            a0, a1, a2, a3 = carry
            a0 = a0 + buf[l, pl.ds(0, LANES)]
            a1 = a1 + buf[l, pl.ds(LANES, LANES)]
            a2 = a2 + buf[l, pl.ds(2 * LANES, LANES)]
            a3 = a3 + buf[l, pl.ds(3 * LANES, LANES)]
            return (a0, a1, a2, a3)

        a0, a1, a2, a3 = lax.fori_loop(0, SEQ, lbody, (zero,) * 4,
                                       unroll=8)
        acc_v[r, pl.ds(0, LANES)] = a0
        acc_v[r, pl.ds(LANES, LANES)] = a1
        acc_v[r, pl.ds(2 * LANES, LANES)] = a2
        acc_v[r, pl.ds(3 * LANES, LANES)] = a3

    def do_row(r, b):
        @pl.when(r + NBUF - 1 < BPW)
        def _():
            _fire_row(table_hbm, idx_v, bufs[(b + NBUF - 1) % NBUF],
                      sems[(b + NBUF - 1) % NBUF], r + NBUF - 1)

        _drain_row(table_hbm, bufs[b], sems[b])
        accum_row(bufs[b], r)

    NG = BPW // NBUF

    def step(g, _):
        for b in range(NBUF):
            do_row(g * NBUF + b, b)
        return 0

    lax.fori_loop(0, NG, step, 0)
    for r in range(NG * NBUF, BPW):
        do_row(r, r % NBUF)

    pltpu.sync_copy(acc_v, out_hbm.at[pl.ds(base, BPW)])


@jax.jit
def _pool(idx, table):
    mesh = plsc.VectorSubcoreMesh(core_axis_name="c", subcore_axis_name="s")
    return pl.kernel(
        _pool_body,
        mesh=mesh,
        compiler_params=pltpu.CompilerParams(use_tc_tiling_on_sc=True),
        out_type=jax.ShapeDtypeStruct((BATCH, DIM), jnp.float32),
        scratch_types=[
            pltpu.VMEM((BPW, SEQ), jnp.int32),
            pltpu.VMEM((SEQ, 128), jnp.float32),
            pltpu.VMEM((SEQ, 128), jnp.float32),
            pltpu.VMEM((SEQ, 128), jnp.float32),
            pltpu.VMEM((BPW, DIM), jnp.float32),
            pltpu.SemaphoreType.DMA,
            pltpu.SemaphoreType.DMA,
            pltpu.SemaphoreType.DMA,
        ],
    )(idx, table)


def _mlp_body(x_ref, w1_ref, b1_ref, w2_ref, b2_ref, o_ref):
    x = x_ref[:] * (1.0 / SEQ)
    h = jnp.dot(x, w1_ref[:], preferred_element_type=jnp.float32) + b1_ref[:]
    z = jnp.dot(h, w2_ref[:], preferred_element_type=jnp.float32) + b2_ref[:]
    z = z - jnp.max(z, axis=-1, keepdims=True)
    e = jnp.exp(z)
    o_ref[:] = e / jnp.sum(e, axis=-1, keepdims=True)


@jax.jit
def _mlp(pooled, W1, b1, W2, b2):
    blk = 1024
    grid = BATCH // blk
    return pl.pallas_call(
        _mlp_body,
        grid=(grid,),
        in_specs=[
            pl.BlockSpec((blk, DIM), lambda i: (i, 0)),
            pl.BlockSpec((DIM, HID), lambda i: (0, 0)),
            pl.BlockSpec((1, HID), lambda i: (0, 0)),
            pl.BlockSpec((HID, OUT), lambda i: (0, 0)),
            pl.BlockSpec((1, OUT), lambda i: (0, 0)),
        ],
        out_specs=pl.BlockSpec((blk, OUT), lambda i: (i, 0)),
        out_shape=jax.ShapeDtypeStruct((BATCH, OUT), jnp.float32),
    )(pooled, W1, b1, W2, b2)


def kernel(inputs, emb_table, W1, b1, W2, b2):
    idx = inputs.astype(jnp.int32)
    table128 = _widen(emb_table.T)
    pooled = _pool(idx, table128)
    return _mlp(pooled, W1, b1.reshape(1, HID), W2, b2.reshape(1, OUT))
